# baseline (speedup 1.0000x reference)
import functools

import jax
import jax.numpy as jnp
from jax import lax
from jax.experimental import pallas as pl
from jax.experimental.pallas import tpu as pltpu
from jax.experimental.pallas import tpu_sc as plsc

LANES = 16
NC = 1
NW = 16 * NC


def _dense_relu_kernel(x_ref, w_ref, b_ref, o_ref):
    acc = jnp.dot(x_ref[...], w_ref[...], preferred_element_type=jnp.float32)
    o_ref[...] = jnp.maximum(acc + b_ref[...], 0.0)


def _dense_relu(x, W, b, block_rows):
    n, d = x.shape
    h = W.shape[1]
    assert n % block_rows == 0
    return pl.pallas_call(
        _dense_relu_kernel,
        grid=(n // block_rows,),
        in_specs=[
            pl.BlockSpec((block_rows, d), lambda i: (i, 0)),
            pl.BlockSpec((d, h), lambda i: (0, 0)),
            pl.BlockSpec((1, h), lambda i: (0, 0)),
        ],
        out_specs=pl.BlockSpec((block_rows, h), lambda i: (i, 0)),
        out_shape=jax.ShapeDtypeStruct((n, h), jnp.float32),
    )(x, W, b.reshape(1, h))


def _make_accumulate(n, K, H, per_w, C):
    assert per_w * NW >= n and per_w % C == 0
    n_chunks = per_w // C
    CK = C * K
    G = CK // 128
    assert G * 128 == CK
    HV = H // LANES
    mesh = plsc.VectorSubcoreMesh(core_axis_name="c", subcore_axis_name="s",
                                  num_cores=NC)

    @functools.partial(
        pl.kernel,
        out_type=jax.ShapeDtypeStruct((n, 2 * H), jnp.float32),
        mesh=mesh,
        compiler_params=pltpu.CompilerParams(use_tc_tiling_on_sc=False),
        scratch_types=[
            pltpu.VMEM((4 * CK,), jnp.int32),
            pltpu.VMEM((4 * CK,), jnp.float32),
            pltpu.VMEM((4 * C, H), jnp.float32),
            pltpu.VMEM((3 * CK, H), jnp.float32),
            pltpu.VMEM((2 * C, 2 * H), jnp.float32),
            pltpu.SemaphoreType.DMA,
            pltpu.SemaphoreType.DMA,
            pltpu.SemaphoreType.DMA,
            pltpu.SemaphoreType.DMA,
            pltpu.SemaphoreType.DMA,
        ],
    )
    def acc(h_hbm, idx_hbm, dsq_hbm, out_hbm,
            idx_v, dsq_v, own_v, rows_v, out_v,
            lin_s, gat_s0, gat_s1, out_s0, out_s1):
        wid = lax.axis_index("s") * NC + lax.axis_index("c")
        base0 = wid * per_w

        def chunk_base(c):
            return jnp.minimum(base0 + c * C, n - C)

        def lin_descs(c):
            base = chunk_base(c)
            o = (c % 4) * CK
            oc = (c % 4) * C
            return (
                (idx_hbm.at[pl.ds(base * K, CK)], idx_v.at[pl.ds(o, CK)]),
                (dsq_hbm.at[pl.ds(base * K, CK)], dsq_v.at[pl.ds(o, CK)]),
                (h_hbm.at[pl.ds(base, C)], own_v.at[pl.ds(oc, C)]),
            )

        def issue_linear(c):
            for src, dst in lin_descs(c):
                pltpu.async_copy(src, dst, lin_s)

        def wait_linear(c):
            for src, dst in lin_descs(c):
                pltpu.make_async_copy(src, dst, lin_s).wait()

        def gat_descs(c):
            oi = (c % 4) * CK
            o = (c % 3) * CK
            return [
                (h_hbm.at[idx_v.at[pl.ds(oi + g * 128, 128)]],
                 rows_v.at[pl.ds(o + g * 128, 128)])
                for g in range(G)
            ]

        def issue_gathers(c, sem):
            for src, dst in gat_descs(c):
                pltpu.async_copy(src, dst, sem)

        def wait_gathers(c, sem):
            for src, dst in gat_descs(c):
                pltpu.make_async_copy(src, dst, sem).wait()

        def out_desc(c, par):
            return (out_v.at[pl.ds(par * C, C)],
                    out_hbm.at[pl.ds(chunk_base(c), C)])

        def compute(c):
            o = (c % 3) * CK
            ol = (c % 4) * CK
            oc = (c % 4) * C
            oo = (c % 2) * C

            @plsc.parallel_loop(0, C, 1, unroll=2)
            def node_body(i):
                wvec = jnp.exp(dsq_v[pl.ds(ol + i * K, K)] * (-10.0))
                rb = o + i * K
                s = [jnp.zeros((LANES,), jnp.float32) for _ in range(HV)]
                m = [jnp.full((LANES,), -jnp.inf, jnp.float32)
                     for _ in range(HV)]
                for k in range(K):
                    wk = wvec[k]
                    for j in range(HV):
                        wg = rows_v[rb + k, pl.ds(j * LANES, LANES)] * wk
                        s[j] = s[j] + wg
                        m[j] = jnp.maximum(m[j], wg)
                for j in range(HV):
                    ow = own_v[oc + i, pl.ds(j * LANES, LANES)]
                    out_v[oo + i, pl.ds(j * LANES, LANES)] = (
                        s[j] * (1.0 / K) - ow)
                    out_v[oo + i, pl.ds(H + j * LANES, LANES)] = m[j] - ow

        issue_linear(0)
        wait_linear(0)
        issue_gathers(0, gat_s0)
        issue_linear(1)
        wait_linear(1)
        issue_gathers(1, gat_s1)
        issue_linear(2)

        def chunk_body(c, carry):
            @pl.when(c < n_chunks - 2)
            def _():
                wait_linear(c + 2)

            for par, gsem in ((0, gat_s0), (1, gat_s1)):
                @pl.when(c % 2 == par)
                def _(par=par, gsem=gsem):
                    wait_gathers(c, gsem)

                    @pl.when(c < n_chunks - 2)
                    def _():
                        issue_gathers(c + 2, gsem)

            @pl.when(c < n_chunks - 3)
            def _():
                issue_linear(c + 3)

            for par, sem in ((0, out_s0), (1, out_s1)):
                @pl.when(jnp.logical_and(c >= 2, c % 2 == par))
                def _(par=par, sem=sem):
                    src, dst = out_desc(c - 2, par)
                    pltpu.make_async_copy(src, dst, sem).wait()

            compute(c)

            for par, sem in ((0, out_s0), (1, out_s1)):
                @pl.when(c % 2 == par)
                def _(par=par, sem=sem):
                    src, dst = out_desc(c, par)
                    pltpu.async_copy(src, dst, sem)
            return carry

        lax.fori_loop(0, n_chunks, chunk_body, 0)
        for c in (n_chunks - 2, n_chunks - 1):
            src, dst = out_desc(c, c % 2)
            pltpu.make_async_copy(src, dst, out_s0 if c % 2 == 0 else out_s1).wait()

    return acc


def kernel(x, neighbor_indices, distancesq, W0, b0, W1, b1):
    n, d = x.shape
    K = neighbor_indices.shape[1]
    H = W0.shape[1]

    C = 32
    per_w = -(-n // (NW * C)) * C

    idx_flat = neighbor_indices.reshape(-1)
    dsq_flat = distancesq.reshape(-1)

    acc = _make_accumulate(n, K, H, per_w, C)

    h0 = _dense_relu(x, W0, b0, block_rows=5000)
    f1 = acc(h0, idx_flat, dsq_flat)
    h1 = _dense_relu(f1, W1, b1, block_rows=5000)
    f2 = acc(h1, idx_flat, dsq_flat)
    return jnp.concatenate([f1, f2, x], axis=-1)


# speedup vs baseline: 1.5805x; 1.5805x over previous
import functools

import jax
import jax.numpy as jnp
from jax import lax
from jax.experimental import pallas as pl
from jax.experimental.pallas import tpu as pltpu
from jax.experimental.pallas import tpu_sc as plsc

LANES = 16
NC = 1
NW = 16 * NC


def _dense_relu_kernel(x_ref, w_ref, b_ref, o_ref):
    acc = jnp.dot(x_ref[...], w_ref[...], preferred_element_type=jnp.float32)
    o_ref[...] = jnp.maximum(acc + b_ref[...], 0.0)


def _dense_relu(x, W, b, block_rows):
    n, d = x.shape
    h = W.shape[1]
    assert n % block_rows == 0
    return pl.pallas_call(
        _dense_relu_kernel,
        grid=(n // block_rows,),
        in_specs=[
            pl.BlockSpec((block_rows, d), lambda i: (i, 0)),
            pl.BlockSpec((d, h), lambda i: (0, 0)),
            pl.BlockSpec((1, h), lambda i: (0, 0)),
        ],
        out_specs=pl.BlockSpec((block_rows, h), lambda i: (i, 0)),
        out_shape=jax.ShapeDtypeStruct((n, h), jnp.float32),
    )(x, W, b.reshape(1, h))


def _make_accumulate(n, K, H, per_w, C):
    assert per_w * NW >= n and per_w % C == 0
    n_chunks = per_w // C
    CK = C * K
    G = CK // 128
    assert G * 128 == CK
    HV = H // LANES
    mesh = plsc.VectorSubcoreMesh(core_axis_name="c", subcore_axis_name="s",
                                  num_cores=NC)

    @functools.partial(
        pl.kernel,
        out_type=jax.ShapeDtypeStruct((n, 2 * H), jnp.float32),
        mesh=mesh,
        compiler_params=pltpu.CompilerParams(use_tc_tiling_on_sc=False),
        scratch_types=[
            pltpu.VMEM((4 * CK,), jnp.int32),
            pltpu.VMEM((4 * CK,), jnp.float32),
            pltpu.VMEM((4 * C, H), jnp.float32),
            pltpu.VMEM((3 * CK, H), jnp.float32),
            pltpu.VMEM((2 * C, 2 * H), jnp.float32),
            pltpu.SemaphoreType.DMA,
            pltpu.SemaphoreType.DMA,
            pltpu.SemaphoreType.DMA,
            pltpu.SemaphoreType.DMA,
            pltpu.SemaphoreType.DMA,
        ],
    )
    def acc(h_hbm, idx_hbm, dsq_hbm, out_hbm,
            idx_v, dsq_v, own_v, rows_v, out_v,
            lin_s, gat_s0, gat_s1, out_s0, out_s1):
        wid = lax.axis_index("s") * NC + lax.axis_index("c")
        base0 = wid * per_w

        def chunk_base(c):
            return jnp.minimum(base0 + c * C, n - C)

        def lin_descs(c):
            base = chunk_base(c)
            o = (c % 4) * CK
            oc = (c % 4) * C
            return (
                (idx_hbm.at[pl.ds(base * K, CK)], idx_v.at[pl.ds(o, CK)]),
                (dsq_hbm.at[pl.ds(base * K, CK)], dsq_v.at[pl.ds(o, CK)]),
                (h_hbm.at[pl.ds(base, C)], own_v.at[pl.ds(oc, C)]),
            )

        def issue_linear(c):
            for src, dst in lin_descs(c):
                pltpu.async_copy(src, dst, lin_s)

        def wait_linear(c):
            for src, dst in lin_descs(c):
                pltpu.make_async_copy(src, dst, lin_s).wait()

        def gat_descs(c):
            oi = (c % 4) * CK
            o = (c % 3) * CK
            return [
                (h_hbm.at[idx_v.at[pl.ds(oi + g * 128, 128)]],
                 rows_v.at[pl.ds(o + g * 128, 128)])
                for g in range(G)
            ]

        def issue_gathers(c, sem):
            for src, dst in gat_descs(c):
                pltpu.async_copy(src, dst, sem)

        def wait_gathers(c, sem):
            for src, dst in gat_descs(c):
                pltpu.make_async_copy(src, dst, sem).wait()

        def out_desc(c, par):
            return (out_v.at[pl.ds(par * C, C)],
                    out_hbm.at[pl.ds(chunk_base(c), C)])

        def compute(c):
            o = (c % 3) * CK
            ol = (c % 4) * CK
            oc = (c % 4) * C
            oo = (c % 2) * C

            @plsc.parallel_loop(0, C, 1)
            def node_body(i):
                wvec = jnp.exp(dsq_v[pl.ds(ol + i * K, K)] * (-10.0))
                rb = o + i * K
                s = [jnp.zeros((LANES,), jnp.float32) for _ in range(HV)]
                m = [jnp.full((LANES,), -jnp.inf, jnp.float32)
                     for _ in range(HV)]
                for k in range(K):
                    wk = wvec[k]
                    for j in range(HV):
                        wg = rows_v[rb + k, pl.ds(j * LANES, LANES)] * wk
                        s[j] = s[j] + wg
                        m[j] = jnp.maximum(m[j], wg)
                for j in range(HV):
                    ow = own_v[oc + i, pl.ds(j * LANES, LANES)]
                    out_v[oo + i, pl.ds(j * LANES, LANES)] = (
                        s[j] * (1.0 / K) - ow)
                    out_v[oo + i, pl.ds(H + j * LANES, LANES)] = m[j] - ow

        issue_linear(0)
        wait_linear(0)
        issue_gathers(0, gat_s0)
        issue_linear(1)
        wait_linear(1)
        issue_gathers(1, gat_s1)
        issue_linear(2)

        def chunk_body(c, carry):
            @pl.when(c < n_chunks - 2)
            def _():
                wait_linear(c + 2)

            for par, gsem in ((0, gat_s0), (1, gat_s1)):
                @pl.when(c % 2 == par)
                def _(par=par, gsem=gsem):
                    wait_gathers(c, gsem)

                    @pl.when(c < n_chunks - 2)
                    def _():
                        issue_gathers(c + 2, gsem)

            @pl.when(c < n_chunks - 3)
            def _():
                issue_linear(c + 3)

            for par, sem in ((0, out_s0), (1, out_s1)):
                @pl.when(jnp.logical_and(c >= 2, c % 2 == par))
                def _(par=par, sem=sem):
                    src, dst = out_desc(c - 2, par)
                    pltpu.make_async_copy(src, dst, sem).wait()

            compute(c)

            for par, sem in ((0, out_s0), (1, out_s1)):
                @pl.when(c % 2 == par)
                def _(par=par, sem=sem):
                    src, dst = out_desc(c, par)
                    pltpu.async_copy(src, dst, sem)
            return carry

        lax.fori_loop(0, n_chunks, chunk_body, 0)
        for c in (n_chunks - 2, n_chunks - 1):
            src, dst = out_desc(c, c % 2)
            pltpu.make_async_copy(src, dst, out_s0 if c % 2 == 0 else out_s1).wait()

    return acc


def kernel(x, neighbor_indices, distancesq, W0, b0, W1, b1):
    n, d = x.shape
    K = neighbor_indices.shape[1]
    H = W0.shape[1]

    C = 32
    per_w = -(-n // (NW * C)) * C

    idx_flat = neighbor_indices.reshape(-1)
    dsq_flat = distancesq.reshape(-1)

    acc = _make_accumulate(n, K, H, per_w, C)

    h0 = _dense_relu(x, W0, b0, block_rows=5000)
    f1 = acc(h0, idx_flat, dsq_flat)
    h1 = _dense_relu(f1, W1, b1, block_rows=5000)
    f2 = acc(h1, idx_flat, dsq_flat)
    return jnp.concatenate([f1, f2, x], axis=-1)
